# baseline (speedup 1.0000x reference)
import functools

import jax
import jax.numpy as jnp
from jax import lax
from jax.experimental import pallas as pl
from jax.experimental.pallas import tpu as pltpu
from jax.experimental.pallas import tpu_sc as plsc

N = 10000
IN_F = 128
OUT_F = 128
ALPHA = 0.2
E = 320000
DP = 144
NC = 2
NS = 16
NW = NC * NS
EC = E // NW
B = 50
NCHUNK = EC // B
RT = N // NS


def _tc1_body(h_ref, w_ref, a2_ref, u_ref, g16_ref):
    wh = jax.nn.leaky_relu(
        jnp.dot(h_ref[...], w_ref[...], preferred_element_type=jnp.float32),
        negative_slope=ALPHA)
    e2 = jnp.sum(wh * a2_ref[...], axis=1, keepdims=True)
    g = jnp.exp(e2 - jnp.max(e2))
    u_ref[...] = wh * g
    lane = lax.broadcasted_iota(jnp.int32, (N, 16), 1)
    g16_ref[...] = jnp.where(lane == 0, g, 0.0)


def _tc1(h, w, a2):
    return pl.pallas_call(
        _tc1_body,
        out_shape=(
            jax.ShapeDtypeStruct((N, OUT_F), jnp.float32),
            jax.ShapeDtypeStruct((N, 16), jnp.float32),
        ),
    )(h, w, a2)


NBUF = 4


def _sc_body(u_hbm, src2_hbm, dst2_hbm, z_hbm, out_hbm,
             src_ring, dst_all, rows_all, acc_sh,
             sem_i, sem_g, sem_sc):
    cid = lax.axis_index("c")
    sid = lax.axis_index("s")
    wid = sid * NC + cid

    pltpu.sync_copy(z_hbm, rows_all.at[0])
    for k in range(RT // B):
        pltpu.sync_copy(rows_all.at[0], acc_sh.at[pl.ds(sid * RT + k * B, B)])
    rem = RT - (RT // B) * B
    if rem:
        pltpu.sync_copy(rows_all.at[0, pl.ds(0, rem)],
                        acc_sh.at[pl.ds(sid * RT + (RT // B) * B, rem)])
    pltpu.sync_copy(dst2_hbm.at[pl.ds(wid * NCHUNK, NCHUNK)], dst_all)
    plsc.subcore_barrier()

    def _fetch(j, buf):
        pltpu.async_copy(src2_hbm.at[wid * NCHUNK + j], src_ring.at[buf],
                         sem_i)
        pltpu.async_copy(u_hbm.at[dst_all.at[j]], rows_all.at[buf], sem_g)

    def _drain(sem, ref):
        pltpu.make_async_copy(u_hbm.at[pl.ds(0, B)], ref, sem).wait()

    _fetch(0, 0)

    def body(j, carry):
        nxt = j + 1

        @pl.when(nxt < NCHUNK)
        def _prefetch():
            @pl.when(nxt >= NBUF)
            def _free_buf():
                _drain(sem_sc, rows_all.at[0])
            _fetch(nxt, lax.rem(nxt, NBUF))

        _drain(sem_g, rows_all.at[0])
        _drain(sem_i, src_ring.at[0])
        buf = lax.rem(j, NBUF)
        pltpu.async_copy(rows_all.at[buf], acc_sh.at[src_ring.at[buf]],
                         sem_sc, add=True)
        return carry

    lax.fori_loop(0, NCHUNK, body, 0)
    for _ in range(NBUF):
        _drain(sem_sc, rows_all.at[0])
    plsc.subcore_barrier()

    for k in range(RT // B + (1 if RT % B else 0)):
        r0 = sid * RT + k * B
        w = min(B, RT - k * B)
        pltpu.sync_copy(acc_sh.at[pl.ds(r0, w)], rows_all.at[0, pl.ds(0, w)])
        pltpu.sync_copy(rows_all.at[0, pl.ds(0, w)],
                        out_hbm.at[cid, pl.ds(r0, w)])


@functools.lru_cache(maxsize=None)
def _sc_agg():
    return pl.kernel(
        _sc_body,
        out_type=jax.ShapeDtypeStruct((NC, N, DP), jnp.float32),
        mesh=plsc.VectorSubcoreMesh(core_axis_name="c", subcore_axis_name="s"),
        compiler_params=pltpu.CompilerParams(use_tc_tiling_on_sc=False),
        scratch_types=[
            pltpu.VMEM((NBUF, B), jnp.int32),
            pltpu.VMEM((NCHUNK, B), jnp.int32),
            pltpu.VMEM((NBUF, B, DP), jnp.float32),
            pltpu.VMEM_SHARED((N, DP), jnp.float32),
            pltpu.SemaphoreType.DMA,
            pltpu.SemaphoreType.DMA,
            pltpu.SemaphoreType.DMA,
        ],
    )


def _tc2_body(acc_ref, out_ref):
    a0 = acc_ref[0]
    a1 = acc_ref[1]
    num = a0[:, :OUT_F] + a1[:, :OUT_F]
    den = jnp.sum(a0[:, OUT_F:] + a1[:, OUT_F:], axis=1, keepdims=True)
    pos = den > 0.0
    hp = jnp.where(pos, num / jnp.where(pos, den, 1.0), 0.0)
    out_ref[...] = jnp.where(hp > 0.0, hp,
                             jnp.exp(jnp.minimum(hp, 0.0)) - 1.0)


def _tc2(acc):
    return pl.pallas_call(
        _tc2_body,
        out_shape=jax.ShapeDtypeStruct((N, OUT_F), jnp.float32),
    )(acc)


def kernel(h, edge_index, W, a):
    a2 = a[OUT_F:, 0][None, :]
    u, g16 = _tc1(h, W, a2)
    u144 = jnp.concatenate([u, g16], axis=1)
    src2 = edge_index[0].reshape(E // B, B)
    dst2 = edge_index[1].reshape(E // B, B)
    z = jnp.zeros((B, DP), jnp.float32)
    acc = _sc_agg()(u144, src2, dst2, z)
    return _tc2(acc)


# speedup vs baseline: 1.1407x; 1.1407x over previous
import functools

import jax
import jax.numpy as jnp
from jax import lax
from jax.experimental import pallas as pl
from jax.experimental.pallas import tpu as pltpu
from jax.experimental.pallas import tpu_sc as plsc

N = 10000
IN_F = 128
OUT_F = 128
ALPHA = 0.2
E = 320000
DP = 144
NC = 2
NS = 16
NW = NC * NS
EC = E // NW
B = 100
NCHUNK = EC // B
RT = N // NS


def _tc1_body(h_ref, w_ref, a2_ref, u_ref, g16_ref):
    wh = jax.nn.leaky_relu(
        jnp.dot(h_ref[...], w_ref[...], preferred_element_type=jnp.float32),
        negative_slope=ALPHA)
    e2 = jnp.sum(wh * a2_ref[...], axis=1, keepdims=True)
    g = jnp.exp(e2 - jnp.max(e2))
    u_ref[...] = wh * g
    lane = lax.broadcasted_iota(jnp.int32, (N, 16), 1)
    g16_ref[...] = jnp.where(lane == 0, g, 0.0)


def _tc1(h, w, a2):
    return pl.pallas_call(
        _tc1_body,
        out_shape=(
            jax.ShapeDtypeStruct((N, OUT_F), jnp.float32),
            jax.ShapeDtypeStruct((N, 16), jnp.float32),
        ),
    )(h, w, a2)


NBUF = 2


def _sc_body(u_hbm, src2_hbm, dst2_hbm, z_hbm, out_hbm,
             src_ring, dst_all, rows_all, acc_sh,
             sem_i, sem_g, sem_sc):
    cid = lax.axis_index("c")
    sid = lax.axis_index("s")
    wid = sid * NC + cid

    pltpu.sync_copy(z_hbm, rows_all.at[0])
    for k in range(RT // B):
        pltpu.sync_copy(rows_all.at[0], acc_sh.at[pl.ds(sid * RT + k * B, B)])
    rem = RT - (RT // B) * B
    if rem:
        pltpu.sync_copy(rows_all.at[0, pl.ds(0, rem)],
                        acc_sh.at[pl.ds(sid * RT + (RT // B) * B, rem)])
    pltpu.sync_copy(dst2_hbm.at[pl.ds(wid * NCHUNK, NCHUNK)], dst_all)
    plsc.subcore_barrier()

    def _fetch(j, buf):
        pltpu.async_copy(src2_hbm.at[wid * NCHUNK + j], src_ring.at[buf],
                         sem_i)
        pltpu.async_copy(u_hbm.at[dst_all.at[j]], rows_all.at[buf], sem_g)

    def _drain(sem, ref):
        pltpu.make_async_copy(u_hbm.at[pl.ds(0, B)], ref, sem).wait()

    _fetch(0, 0)

    def body(j, carry):
        nxt = j + 1

        @pl.when(nxt < NCHUNK)
        def _prefetch():
            @pl.when(nxt >= NBUF)
            def _free_buf():
                _drain(sem_sc, rows_all.at[0])
            _fetch(nxt, lax.rem(nxt, NBUF))

        _drain(sem_g, rows_all.at[0])
        _drain(sem_i, src_ring.at[0])
        buf = lax.rem(j, NBUF)
        pltpu.async_copy(rows_all.at[buf], acc_sh.at[src_ring.at[buf]],
                         sem_sc, add=True)
        return carry

    lax.fori_loop(0, NCHUNK, body, 0)
    for _ in range(NBUF):
        _drain(sem_sc, rows_all.at[0])
    plsc.subcore_barrier()

    for k in range(RT // B + (1 if RT % B else 0)):
        r0 = sid * RT + k * B
        w = min(B, RT - k * B)
        pltpu.sync_copy(acc_sh.at[pl.ds(r0, w)], rows_all.at[0, pl.ds(0, w)])
        pltpu.sync_copy(rows_all.at[0, pl.ds(0, w)],
                        out_hbm.at[cid, pl.ds(r0, w)])


@functools.lru_cache(maxsize=None)
def _sc_agg():
    return pl.kernel(
        _sc_body,
        out_type=jax.ShapeDtypeStruct((NC, N, DP), jnp.float32),
        mesh=plsc.VectorSubcoreMesh(core_axis_name="c", subcore_axis_name="s"),
        compiler_params=pltpu.CompilerParams(use_tc_tiling_on_sc=False),
        scratch_types=[
            pltpu.VMEM((NBUF, B), jnp.int32),
            pltpu.VMEM((NCHUNK, B), jnp.int32),
            pltpu.VMEM((NBUF, B, DP), jnp.float32),
            pltpu.VMEM_SHARED((N, DP), jnp.float32),
            pltpu.SemaphoreType.DMA,
            pltpu.SemaphoreType.DMA,
            pltpu.SemaphoreType.DMA,
        ],
    )


def _tc2_body(acc_ref, out_ref):
    a0 = acc_ref[0]
    a1 = acc_ref[1]
    num = a0[:, :OUT_F] + a1[:, :OUT_F]
    den = jnp.sum(a0[:, OUT_F:] + a1[:, OUT_F:], axis=1, keepdims=True)
    pos = den > 0.0
    hp = jnp.where(pos, num / jnp.where(pos, den, 1.0), 0.0)
    out_ref[...] = jnp.where(hp > 0.0, hp,
                             jnp.exp(jnp.minimum(hp, 0.0)) - 1.0)


def _tc2(acc):
    return pl.pallas_call(
        _tc2_body,
        out_shape=jax.ShapeDtypeStruct((N, OUT_F), jnp.float32),
    )(acc)


def kernel(h, edge_index, W, a):
    a2 = a[OUT_F:, 0][None, :]
    u, g16 = _tc1(h, W, a2)
    u144 = jnp.concatenate([u, g16], axis=1)
    src2 = edge_index[0].reshape(E // B, B)
    dst2 = edge_index[1].reshape(E // B, B)
    z = jnp.zeros((B, DP), jnp.float32)
    acc = _sc_agg()(u144, src2, dst2, z)
    return _tc2(acc)


# speedup vs baseline: 1.3661x; 1.1975x over previous
import functools

import jax
import jax.numpy as jnp
from jax import lax
from jax.experimental import pallas as pl
from jax.experimental.pallas import tpu as pltpu
from jax.experimental.pallas import tpu_sc as plsc

N = 10000
IN_F = 128
OUT_F = 128
ALPHA = 0.2
E = 320000
DG = 16
NC = 2
NS = 16
NW = NC * NS
EC = E // NW
B = 80
NCHUNK = EC // B
RT = N // NS


def _tc1_body(h_ref, w_ref, a2_ref, u_ref, g16_ref):
    wh = jax.nn.leaky_relu(
        jnp.dot(h_ref[...], w_ref[...], preferred_element_type=jnp.float32),
        negative_slope=ALPHA)
    e2 = jnp.sum(wh * a2_ref[...], axis=1, keepdims=True)
    g = jnp.exp(e2 - jnp.max(e2))
    u_ref[...] = wh * g
    lane = lax.broadcasted_iota(jnp.int32, (N, DG), 1)
    g16_ref[...] = jnp.where(lane == 0, g, 0.0)


def _tc1(h, w, a2):
    return pl.pallas_call(
        _tc1_body,
        out_shape=(
            jax.ShapeDtypeStruct((N, OUT_F), jnp.float32),
            jax.ShapeDtypeStruct((N, DG), jnp.float32),
        ),
    )(h, w, a2)


NBUF = 2


def _sc_body(u_hbm, g16_hbm, ei_hbm, zn_hbm, zd_hbm, out_num, out_den,
             src_ring, dst_all, rows_n, rows_d, accn_sh, accd_sh,
             sem_i, sem_g, sem_sc):
    cid = lax.axis_index("c")
    sid = lax.axis_index("s")
    wid = sid * NC + cid
    ebase = wid * EC

    pltpu.sync_copy(zn_hbm, rows_n.at[0])
    pltpu.sync_copy(zd_hbm, rows_d.at[0])
    nfull = RT // B
    for k in range(nfull):
        pltpu.sync_copy(rows_n.at[0], accn_sh.at[pl.ds(sid * RT + k * B, B)])
        pltpu.sync_copy(rows_d.at[0], accd_sh.at[pl.ds(sid * RT + k * B, B)])
    rem = RT - nfull * B
    if rem:
        r0 = sid * RT + nfull * B
        pltpu.sync_copy(rows_n.at[0, pl.ds(0, rem)],
                        accn_sh.at[pl.ds(r0, rem)])
        pltpu.sync_copy(rows_d.at[0, pl.ds(0, rem)],
                        accd_sh.at[pl.ds(r0, rem)])
    pltpu.sync_copy(ei_hbm.at[1, pl.ds(ebase, EC)], dst_all)
    plsc.subcore_barrier()

    def _fetch(j, buf):
        pltpu.async_copy(ei_hbm.at[0, pl.ds(ebase + j * B, B)],
                         src_ring.at[buf], sem_i)
        idx = dst_all.at[pl.ds(j * B, B)]
        pltpu.async_copy(u_hbm.at[idx], rows_n.at[buf], sem_g)
        pltpu.async_copy(g16_hbm.at[idx], rows_d.at[buf], sem_g)

    def _drain(sem, ref):
        pltpu.make_async_copy(u_hbm.at[pl.ds(0, B)], ref, sem).wait()

    _fetch(0, 0)

    def body(j, carry):
        nxt = j + 1

        @pl.when(nxt < NCHUNK)
        def _prefetch():
            @pl.when(nxt >= NBUF)
            def _free_buf():
                _drain(sem_sc, rows_n.at[0])
                _drain(sem_sc, rows_d.at[0])
            _fetch(nxt, lax.rem(nxt, NBUF))

        _drain(sem_g, rows_n.at[0])
        _drain(sem_g, rows_d.at[0])
        _drain(sem_i, src_ring.at[0])
        buf = lax.rem(j, NBUF)
        idx = src_ring.at[buf]
        pltpu.async_copy(rows_n.at[buf], accn_sh.at[idx], sem_sc, add=True)
        pltpu.async_copy(rows_d.at[buf], accd_sh.at[idx], sem_sc, add=True)
        return carry

    lax.fori_loop(0, NCHUNK, body, 0)
    for _ in range(NBUF):
        _drain(sem_sc, rows_n.at[0])
        _drain(sem_sc, rows_d.at[0])
    plsc.subcore_barrier()

    for k in range(nfull + (1 if rem else 0)):
        r0 = sid * RT + k * B
        w = min(B, RT - k * B)
        pltpu.sync_copy(accn_sh.at[pl.ds(r0, w)], rows_n.at[0, pl.ds(0, w)])
        pltpu.sync_copy(rows_n.at[0, pl.ds(0, w)],
                        out_num.at[cid, pl.ds(r0, w)])
        pltpu.sync_copy(accd_sh.at[pl.ds(r0, w)], rows_d.at[0, pl.ds(0, w)])
        pltpu.sync_copy(rows_d.at[0, pl.ds(0, w)],
                        out_den.at[cid, pl.ds(r0, w)])


@functools.lru_cache(maxsize=None)
def _sc_agg():
    return pl.kernel(
        _sc_body,
        out_type=(
            jax.ShapeDtypeStruct((NC, N, OUT_F), jnp.float32),
            jax.ShapeDtypeStruct((NC, N, DG), jnp.float32),
        ),
        mesh=plsc.VectorSubcoreMesh(core_axis_name="c", subcore_axis_name="s"),
        compiler_params=pltpu.CompilerParams(use_tc_tiling_on_sc=False),
        scratch_types=[
            pltpu.VMEM((NBUF, B), jnp.int32),
            pltpu.VMEM((EC,), jnp.int32),
            pltpu.VMEM((NBUF, B, OUT_F), jnp.float32),
            pltpu.VMEM((NBUF, B, DG), jnp.float32),
            pltpu.VMEM_SHARED((N, OUT_F), jnp.float32),
            pltpu.VMEM_SHARED((N, DG), jnp.float32),
            pltpu.SemaphoreType.DMA,
            pltpu.SemaphoreType.DMA,
            pltpu.SemaphoreType.DMA,
        ],
    )


def _tc2_body(num_ref, den_ref, out_ref):
    num = num_ref[0] + num_ref[1]
    den = jnp.sum(den_ref[0] + den_ref[1], axis=1, keepdims=True)
    pos = den > 0.0
    hp = jnp.where(pos, num / jnp.where(pos, den, 1.0), 0.0)
    out_ref[...] = jnp.where(hp > 0.0, hp,
                             jnp.exp(jnp.minimum(hp, 0.0)) - 1.0)


def _tc2(num, den):
    return pl.pallas_call(
        _tc2_body,
        out_shape=jax.ShapeDtypeStruct((N, OUT_F), jnp.float32),
    )(num, den)


def kernel(h, edge_index, W, a):
    a2 = a[OUT_F:, 0][None, :]
    u, g16 = _tc1(h, W, a2)
    zn = jnp.zeros((B, OUT_F), jnp.float32)
    zd = jnp.zeros((B, DG), jnp.float32)
    num, den = _sc_agg()(u, g16, edge_index, zn, zd)
    return _tc2(num, den)


# speedup vs baseline: 1.3934x; 1.0200x over previous
import functools

import jax
import jax.numpy as jnp
from jax import lax
from jax.experimental import pallas as pl
from jax.experimental.pallas import tpu as pltpu
from jax.experimental.pallas import tpu_sc as plsc

N = 10000
IN_F = 128
OUT_F = 128
ALPHA = 0.2
E = 320000
DG = 16
NC = 2
NS = 16
NW = NC * NS
EC = E // NW
B = 80
NCHUNK = EC // B
RT = N // NS


def _tc1_body(h_ref, w_ref, a2_ref, u_ref, g16_ref):
    wh = jax.nn.leaky_relu(
        jnp.dot(h_ref[...], w_ref[...], preferred_element_type=jnp.float32),
        negative_slope=ALPHA)
    e2 = jnp.sum(wh * a2_ref[...], axis=1, keepdims=True)
    g = jnp.exp(e2 - jnp.max(e2))
    u_ref[...] = (wh * g).astype(jnp.bfloat16)
    lane = lax.broadcasted_iota(jnp.int32, (N, DG), 1)
    g16_ref[...] = jnp.where(lane == 0, g, 0.0)


def _tc1(h, w, a2):
    return pl.pallas_call(
        _tc1_body,
        out_shape=(
            jax.ShapeDtypeStruct((N, OUT_F), jnp.bfloat16),
            jax.ShapeDtypeStruct((N, DG), jnp.float32),
        ),
    )(h, w, a2)


NBUF = 2


def _sc_body(u_hbm, g16_hbm, ei_hbm, zn_hbm, zd_hbm, out_num, out_den,
             src_ring, dst_all, rows_n, rows_d, accn_sh, accd_sh,
             sem_i, sem_g, sem_sc):
    cid = lax.axis_index("c")
    sid = lax.axis_index("s")
    wid = sid * NC + cid
    ebase = wid * EC

    pltpu.sync_copy(zn_hbm, rows_n.at[0])
    pltpu.sync_copy(zd_hbm, rows_d.at[0])
    nfull = RT // B
    for k in range(nfull):
        pltpu.sync_copy(rows_n.at[0], accn_sh.at[pl.ds(sid * RT + k * B, B)])
        pltpu.sync_copy(rows_d.at[0], accd_sh.at[pl.ds(sid * RT + k * B, B)])
    rem = RT - nfull * B
    if rem:
        r0 = sid * RT + nfull * B
        pltpu.sync_copy(rows_n.at[0, pl.ds(0, rem)],
                        accn_sh.at[pl.ds(r0, rem)])
        pltpu.sync_copy(rows_d.at[0, pl.ds(0, rem)],
                        accd_sh.at[pl.ds(r0, rem)])
    pltpu.sync_copy(ei_hbm.at[1, pl.ds(ebase, EC)], dst_all)
    plsc.subcore_barrier()

    def _fetch(j, buf):
        pltpu.async_copy(ei_hbm.at[0, pl.ds(ebase + j * B, B)],
                         src_ring.at[buf], sem_i)
        idx = dst_all.at[pl.ds(j * B, B)]
        pltpu.async_copy(u_hbm.at[idx], rows_n.at[buf], sem_g)
        pltpu.async_copy(g16_hbm.at[idx], rows_d.at[buf], sem_g)

    def _drain(sem, ref):
        pltpu.make_async_copy(u_hbm.at[pl.ds(0, B)], ref, sem).wait()

    _fetch(0, 0)

    def body(j, carry):
        nxt = j + 1

        @pl.when(nxt < NCHUNK)
        def _prefetch():
            @pl.when(nxt >= NBUF)
            def _free_buf():
                _drain(sem_sc, rows_n.at[0])
                _drain(sem_sc, rows_d.at[0])
            _fetch(nxt, lax.rem(nxt, NBUF))

        _drain(sem_g, rows_n.at[0])
        _drain(sem_g, rows_d.at[0])
        _drain(sem_i, src_ring.at[0])
        buf = lax.rem(j, NBUF)
        idx = src_ring.at[buf]
        pltpu.async_copy(rows_n.at[buf], accn_sh.at[idx], sem_sc, add=True)
        pltpu.async_copy(rows_d.at[buf], accd_sh.at[idx], sem_sc, add=True)
        return carry

    lax.fori_loop(0, NCHUNK, body, 0)
    for _ in range(NBUF):
        _drain(sem_sc, rows_n.at[0])
        _drain(sem_sc, rows_d.at[0])
    plsc.subcore_barrier()

    for k in range(nfull + (1 if rem else 0)):
        r0 = sid * RT + k * B
        w = min(B, RT - k * B)
        pltpu.sync_copy(accn_sh.at[pl.ds(r0, w)], rows_n.at[0, pl.ds(0, w)])
        pltpu.sync_copy(rows_n.at[0, pl.ds(0, w)],
                        out_num.at[cid, pl.ds(r0, w)])
        pltpu.sync_copy(accd_sh.at[pl.ds(r0, w)], rows_d.at[0, pl.ds(0, w)])
        pltpu.sync_copy(rows_d.at[0, pl.ds(0, w)],
                        out_den.at[cid, pl.ds(r0, w)])


@functools.lru_cache(maxsize=None)
def _sc_agg():
    return pl.kernel(
        _sc_body,
        out_type=(
            jax.ShapeDtypeStruct((NC, N, OUT_F), jnp.bfloat16),
            jax.ShapeDtypeStruct((NC, N, DG), jnp.float32),
        ),
        mesh=plsc.VectorSubcoreMesh(core_axis_name="c", subcore_axis_name="s"),
        compiler_params=pltpu.CompilerParams(use_tc_tiling_on_sc=False),
        scratch_types=[
            pltpu.VMEM((NBUF, B), jnp.int32),
            pltpu.VMEM((EC,), jnp.int32),
            pltpu.VMEM((NBUF, B, OUT_F), jnp.bfloat16),
            pltpu.VMEM((NBUF, B, DG), jnp.float32),
            pltpu.VMEM_SHARED((N, OUT_F), jnp.bfloat16),
            pltpu.VMEM_SHARED((N, DG), jnp.float32),
            pltpu.SemaphoreType.DMA,
            pltpu.SemaphoreType.DMA,
            pltpu.SemaphoreType.DMA,
        ],
    )


def _tc2_body(num_ref, den_ref, out_ref):
    num = (num_ref[0].astype(jnp.float32) + num_ref[1].astype(jnp.float32))
    den = jnp.sum(den_ref[0] + den_ref[1], axis=1, keepdims=True)
    pos = den > 0.0
    hp = jnp.where(pos, num / jnp.where(pos, den, 1.0), 0.0)
    out_ref[...] = jnp.where(hp > 0.0, hp,
                             jnp.exp(jnp.minimum(hp, 0.0)) - 1.0)


def _tc2(num, den):
    return pl.pallas_call(
        _tc2_body,
        out_shape=jax.ShapeDtypeStruct((N, OUT_F), jnp.float32),
    )(num, den)


def kernel(h, edge_index, W, a):
    a2 = a[OUT_F:, 0][None, :]
    u, g16 = _tc1(h, W, a2)
    zn = jnp.zeros((B, OUT_F), jnp.bfloat16)
    zd = jnp.zeros((B, DG), jnp.float32)
    num, den = _sc_agg()(u, g16, edge_index, zn, zd)
    return _tc2(num, den)


# speedup vs baseline: 1.4851x; 1.0658x over previous
import functools

import jax
import jax.numpy as jnp
from jax import lax
from jax.experimental import pallas as pl
from jax.experimental.pallas import tpu as pltpu
from jax.experimental.pallas import tpu_sc as plsc

N = 10000
IN_F = 128
OUT_F = 128
ALPHA = 0.2
E = 320000
DG = 16
NC = 2
NS = 16
NW = NC * NS
EC = E // NW
B = 80
NCHUNK = EC // B
RT = N // NS


def _tc1_body(h_ref, w_ref, a2_ref, u_ref, g16_ref):
    wh = jax.nn.leaky_relu(
        jnp.dot(h_ref[...], w_ref[...], preferred_element_type=jnp.float32),
        negative_slope=ALPHA)
    e2 = jnp.sum(wh * a2_ref[...], axis=1, keepdims=True)
    g = jnp.exp(e2 - jnp.max(e2))
    u_ref[...] = (wh * g).astype(jnp.bfloat16)
    lane = lax.broadcasted_iota(jnp.int32, (N, DG), 1)
    g16_ref[...] = jnp.where(lane == 0, g, 0.0)


def _tc1(h, w, a2):
    return pl.pallas_call(
        _tc1_body,
        out_shape=(
            jax.ShapeDtypeStruct((N, OUT_F), jnp.bfloat16),
            jax.ShapeDtypeStruct((N, DG), jnp.float32),
        ),
    )(h, w, a2)


NBUF = 4


def _sc_body(u_hbm, g16_hbm, ei_hbm, zn_hbm, zd_hbm, out_num, out_den,
             src_ring, dst_all, rows_n, rows_d, accn_sh, accd_sh,
             sem_i, sem_g, sem_sc):
    cid = lax.axis_index("c")
    sid = lax.axis_index("s")
    wid = sid * NC + cid
    ebase = wid * EC

    pltpu.sync_copy(zn_hbm, rows_n.at[0])
    pltpu.sync_copy(zd_hbm, rows_d.at[0])
    nfull = RT // B
    for k in range(nfull):
        pltpu.sync_copy(rows_n.at[0], accn_sh.at[pl.ds(sid * RT + k * B, B)])
        pltpu.sync_copy(rows_d.at[0], accd_sh.at[pl.ds(sid * RT + k * B, B)])
    rem = RT - nfull * B
    if rem:
        r0 = sid * RT + nfull * B
        pltpu.sync_copy(rows_n.at[0, pl.ds(0, rem)],
                        accn_sh.at[pl.ds(r0, rem)])
        pltpu.sync_copy(rows_d.at[0, pl.ds(0, rem)],
                        accd_sh.at[pl.ds(r0, rem)])
    pltpu.sync_copy(ei_hbm.at[1, pl.ds(ebase, EC)], dst_all)
    plsc.subcore_barrier()

    def _fetch(j, buf):
        pltpu.async_copy(ei_hbm.at[0, pl.ds(ebase + j * B, B)],
                         src_ring.at[buf], sem_i)
        idx = dst_all.at[pl.ds(j * B, B)]
        pltpu.async_copy(u_hbm.at[idx], rows_n.at[buf], sem_g)
        pltpu.async_copy(g16_hbm.at[idx], rows_d.at[buf], sem_g)

    def _drain(sem, ref):
        pltpu.make_async_copy(u_hbm.at[pl.ds(0, B)], ref, sem).wait()

    _fetch(0, 0)

    def body(j, carry):
        nxt = j + 1

        @pl.when(nxt < NCHUNK)
        def _prefetch():
            @pl.when(nxt >= NBUF)
            def _free_buf():
                _drain(sem_sc, rows_n.at[0])
                _drain(sem_sc, rows_d.at[0])
            _fetch(nxt, lax.rem(nxt, NBUF))

        _drain(sem_g, rows_n.at[0])
        _drain(sem_g, rows_d.at[0])
        _drain(sem_i, src_ring.at[0])
        buf = lax.rem(j, NBUF)
        idx = src_ring.at[buf]
        pltpu.async_copy(rows_n.at[buf], accn_sh.at[idx], sem_sc, add=True)
        pltpu.async_copy(rows_d.at[buf], accd_sh.at[idx], sem_sc, add=True)
        return carry

    lax.fori_loop(0, NCHUNK, body, 0)
    for _ in range(NBUF):
        _drain(sem_sc, rows_n.at[0])
        _drain(sem_sc, rows_d.at[0])
    plsc.subcore_barrier()

    for k in range(nfull + (1 if rem else 0)):
        r0 = sid * RT + k * B
        w = min(B, RT - k * B)
        pltpu.sync_copy(accn_sh.at[pl.ds(r0, w)], rows_n.at[0, pl.ds(0, w)])
        pltpu.sync_copy(rows_n.at[0, pl.ds(0, w)],
                        out_num.at[cid, pl.ds(r0, w)])
        pltpu.sync_copy(accd_sh.at[pl.ds(r0, w)], rows_d.at[0, pl.ds(0, w)])
        pltpu.sync_copy(rows_d.at[0, pl.ds(0, w)],
                        out_den.at[cid, pl.ds(r0, w)])


@functools.lru_cache(maxsize=None)
def _sc_agg():
    return pl.kernel(
        _sc_body,
        out_type=(
            jax.ShapeDtypeStruct((NC, N, OUT_F), jnp.bfloat16),
            jax.ShapeDtypeStruct((NC, N, DG), jnp.float32),
        ),
        mesh=plsc.VectorSubcoreMesh(core_axis_name="c", subcore_axis_name="s"),
        compiler_params=pltpu.CompilerParams(use_tc_tiling_on_sc=False),
        scratch_types=[
            pltpu.VMEM((NBUF, B), jnp.int32),
            pltpu.VMEM((EC,), jnp.int32),
            pltpu.VMEM((NBUF, B, OUT_F), jnp.bfloat16),
            pltpu.VMEM((NBUF, B, DG), jnp.float32),
            pltpu.VMEM_SHARED((N, OUT_F), jnp.bfloat16),
            pltpu.VMEM_SHARED((N, DG), jnp.float32),
            pltpu.SemaphoreType.DMA,
            pltpu.SemaphoreType.DMA,
            pltpu.SemaphoreType.DMA,
        ],
    )


def _tc2_body(num_ref, den_ref, out_ref):
    num = (num_ref[0].astype(jnp.float32) + num_ref[1].astype(jnp.float32))
    den = jnp.sum(den_ref[0] + den_ref[1], axis=1, keepdims=True)
    pos = den > 0.0
    hp = jnp.where(pos, num / jnp.where(pos, den, 1.0), 0.0)
    out_ref[...] = jnp.where(hp > 0.0, hp,
                             jnp.exp(jnp.minimum(hp, 0.0)) - 1.0)


def _tc2(num, den):
    return pl.pallas_call(
        _tc2_body,
        out_shape=jax.ShapeDtypeStruct((N, OUT_F), jnp.float32),
    )(num, den)


def kernel(h, edge_index, W, a):
    a2 = a[OUT_F:, 0][None, :]
    u, g16 = _tc1(h, W, a2)
    zn = jnp.zeros((B, OUT_F), jnp.bfloat16)
    zd = jnp.zeros((B, DG), jnp.float32)
    num, den = _sc_agg()(u, g16, edge_index, zn, zd)
    return _tc2(num, den)


# speedup vs baseline: 1.4864x; 1.0008x over previous
import functools

import jax
import jax.numpy as jnp
from jax import lax
from jax.experimental import pallas as pl
from jax.experimental.pallas import tpu as pltpu
from jax.experimental.pallas import tpu_sc as plsc

N = 10000
IN_F = 128
OUT_F = 128
ALPHA = 0.2
E = 320000
DG = 16
NC = 2
NS = 16
NW = NC * NS
EC = E // NW
B = 80
NCHUNK = EC // B
RT = N // NS


def _tc1_body(h_ref, w_ref, a2_ref, u_ref, g16_ref):
    wh = jax.nn.leaky_relu(
        jnp.dot(h_ref[...], w_ref[...], preferred_element_type=jnp.float32),
        negative_slope=ALPHA)
    e2 = jnp.sum(wh * a2_ref[...], axis=1, keepdims=True)
    g = jnp.exp(e2 - jnp.max(e2))
    u_ref[...] = (wh * g).astype(jnp.bfloat16)
    lane = lax.broadcasted_iota(jnp.int32, (N, DG), 1)
    g16_ref[...] = jnp.where(lane == 0, g, 0.0)


def _tc1(h, w, a2):
    return pl.pallas_call(
        _tc1_body,
        out_shape=(
            jax.ShapeDtypeStruct((N, OUT_F), jnp.bfloat16),
            jax.ShapeDtypeStruct((N, DG), jnp.float32),
        ),
    )(h, w, a2)


NBUF = 6


def _sc_body(u_hbm, g16_hbm, ei_hbm, zn_hbm, zd_hbm, out_num, out_den,
             src_ring, dst_all, rows_n, rows_d, accn_sh, accd_sh,
             sem_i, sem_g, sem_sc):
    cid = lax.axis_index("c")
    sid = lax.axis_index("s")
    wid = sid * NC + cid
    ebase = wid * EC

    pltpu.sync_copy(zn_hbm, rows_n.at[0])
    pltpu.sync_copy(zd_hbm, rows_d.at[0])
    nfull = RT // B
    for k in range(nfull):
        pltpu.sync_copy(rows_n.at[0], accn_sh.at[pl.ds(sid * RT + k * B, B)])
        pltpu.sync_copy(rows_d.at[0], accd_sh.at[pl.ds(sid * RT + k * B, B)])
    rem = RT - nfull * B
    if rem:
        r0 = sid * RT + nfull * B
        pltpu.sync_copy(rows_n.at[0, pl.ds(0, rem)],
                        accn_sh.at[pl.ds(r0, rem)])
        pltpu.sync_copy(rows_d.at[0, pl.ds(0, rem)],
                        accd_sh.at[pl.ds(r0, rem)])
    pltpu.sync_copy(ei_hbm.at[1, pl.ds(ebase, EC)], dst_all)
    plsc.subcore_barrier()

    def _fetch(j, buf):
        pltpu.async_copy(ei_hbm.at[0, pl.ds(ebase + j * B, B)],
                         src_ring.at[buf], sem_i)
        idx = dst_all.at[pl.ds(j * B, B)]
        pltpu.async_copy(u_hbm.at[idx], rows_n.at[buf], sem_g)
        pltpu.async_copy(g16_hbm.at[idx], rows_d.at[buf], sem_g)

    def _drain(sem, ref):
        pltpu.make_async_copy(u_hbm.at[pl.ds(0, B)], ref, sem).wait()

    _fetch(0, 0)

    def body(j, carry):
        nxt = j + 1

        @pl.when(nxt < NCHUNK)
        def _prefetch():
            @pl.when(nxt >= NBUF)
            def _free_buf():
                _drain(sem_sc, rows_n.at[0])
                _drain(sem_sc, rows_d.at[0])
            _fetch(nxt, lax.rem(nxt, NBUF))

        _drain(sem_g, rows_n.at[0])
        _drain(sem_g, rows_d.at[0])
        _drain(sem_i, src_ring.at[0])
        buf = lax.rem(j, NBUF)
        idx = src_ring.at[buf]
        pltpu.async_copy(rows_n.at[buf], accn_sh.at[idx], sem_sc, add=True)
        pltpu.async_copy(rows_d.at[buf], accd_sh.at[idx], sem_sc, add=True)
        return carry

    lax.fori_loop(0, NCHUNK, body, 0)
    for _ in range(NBUF):
        _drain(sem_sc, rows_n.at[0])
        _drain(sem_sc, rows_d.at[0])
    plsc.subcore_barrier()

    for k in range(nfull + (1 if rem else 0)):
        r0 = sid * RT + k * B
        w = min(B, RT - k * B)
        pltpu.sync_copy(accn_sh.at[pl.ds(r0, w)], rows_n.at[0, pl.ds(0, w)])
        pltpu.sync_copy(rows_n.at[0, pl.ds(0, w)],
                        out_num.at[cid, pl.ds(r0, w)])
        pltpu.sync_copy(accd_sh.at[pl.ds(r0, w)], rows_d.at[0, pl.ds(0, w)])
        pltpu.sync_copy(rows_d.at[0, pl.ds(0, w)],
                        out_den.at[cid, pl.ds(r0, w)])


@functools.lru_cache(maxsize=None)
def _sc_agg():
    return pl.kernel(
        _sc_body,
        out_type=(
            jax.ShapeDtypeStruct((NC, N, OUT_F), jnp.bfloat16),
            jax.ShapeDtypeStruct((NC, N, DG), jnp.float32),
        ),
        mesh=plsc.VectorSubcoreMesh(core_axis_name="c", subcore_axis_name="s"),
        compiler_params=pltpu.CompilerParams(use_tc_tiling_on_sc=False),
        scratch_types=[
            pltpu.VMEM((NBUF, B), jnp.int32),
            pltpu.VMEM((EC,), jnp.int32),
            pltpu.VMEM((NBUF, B, OUT_F), jnp.bfloat16),
            pltpu.VMEM((NBUF, B, DG), jnp.float32),
            pltpu.VMEM_SHARED((N, OUT_F), jnp.bfloat16),
            pltpu.VMEM_SHARED((N, DG), jnp.float32),
            pltpu.SemaphoreType.DMA,
            pltpu.SemaphoreType.DMA,
            pltpu.SemaphoreType.DMA,
        ],
    )


def _tc2_body(num_ref, den_ref, out_ref):
    num = (num_ref[0].astype(jnp.float32) + num_ref[1].astype(jnp.float32))
    den = jnp.sum(den_ref[0] + den_ref[1], axis=1, keepdims=True)
    pos = den > 0.0
    hp = jnp.where(pos, num / jnp.where(pos, den, 1.0), 0.0)
    out_ref[...] = jnp.where(hp > 0.0, hp,
                             jnp.exp(jnp.minimum(hp, 0.0)) - 1.0)


def _tc2(num, den):
    return pl.pallas_call(
        _tc2_body,
        out_shape=jax.ShapeDtypeStruct((N, OUT_F), jnp.float32),
    )(num, den)


def kernel(h, edge_index, W, a):
    a2 = a[OUT_F:, 0][None, :]
    u, g16 = _tc1(h, W, a2)
    zn = jnp.zeros((B, OUT_F), jnp.bfloat16)
    zd = jnp.zeros((B, DG), jnp.float32)
    num, den = _sc_agg()(u, g16, edge_index, zn, zd)
    return _tc2(num, den)


# speedup vs baseline: 1.5390x; 1.0354x over previous
import functools

import jax
import jax.numpy as jnp
from jax import lax
from jax.experimental import pallas as pl
from jax.experimental.pallas import tpu as pltpu
from jax.experimental.pallas import tpu_sc as plsc

N = 10000
IN_F = 128
OUT_F = 128
ALPHA = 0.2
E = 320000
DG = 16
NC = 2
NS = 16
NW = NC * NS
EC = E // NW
B = 80
NCHUNK = EC // B
RT = N // NS
RTA = 632
NBLA = (RTA + 15) // 16


def _tc1_body(h_ref, w_ref, a2_ref, u_ref, g16_ref):
    wh = jax.nn.leaky_relu(
        jnp.dot(h_ref[...], w_ref[...], preferred_element_type=jnp.float32),
        negative_slope=ALPHA)
    e2 = jnp.sum(wh * a2_ref[...], axis=1, keepdims=True)
    g = jnp.exp(e2 - jnp.max(e2))
    u_ref[...] = (wh * g).astype(jnp.bfloat16)
    lane = lax.broadcasted_iota(jnp.int32, (N, DG), 1)
    g16_ref[...] = jnp.where(lane == 0, g, 0.0)


def _tc1(h, w, a2):
    return pl.pallas_call(
        _tc1_body,
        out_shape=(
            jax.ShapeDtypeStruct((N, OUT_F), jnp.bfloat16),
            jax.ShapeDtypeStruct((N, DG), jnp.float32),
        ),
    )(h, w, a2)


NBUF = 3
LQ = 16


def _sc_body(u_hbm, g16_hbm, ei_hbm, zn_hbm, out_num, out_den,
             src_ring, dst_all, rows_n, gbuf, gpart, g_vmem, den_local,
             den_stage, dred, accn_sh, g_sh, den_all,
             sem_i, sem_g, sem_sc):
    cid = lax.axis_index("c")
    sid = lax.axis_index("s")
    wid = sid * NC + cid
    ebase = wid * EC
    rbase = sid * RT
    ra = pl.multiple_of(rbase - lax.rem(rbase, 8), 8)
    iota = lax.iota(jnp.int32, LQ)
    zeros16 = jnp.zeros((LQ,), jnp.float32)
    zidx = jnp.zeros((LQ,), jnp.int32)

    pltpu.sync_copy(zn_hbm, rows_n.at[0])
    nfull = RT // B
    for k in range(nfull):
        pltpu.sync_copy(rows_n.at[0], accn_sh.at[pl.ds(rbase + k * B, B)])
    rem = RT - nfull * B
    if rem:
        pltpu.sync_copy(rows_n.at[0, pl.ds(0, rem)],
                        accn_sh.at[pl.ds(rbase + nfull * B, rem)])
    pltpu.sync_copy(ei_hbm.at[1, pl.ds(ebase, EC)], dst_all)

    pltpu.sync_copy(g16_hbm.at[pl.ds(ra, RTA)], gbuf.at[pl.ds(0, RTA)])

    def _extract(kk, carry):
        gv = plsc.load_gather(gbuf, [iota + kk * LQ, zidx])
        gpart[pl.ds(kk * LQ, LQ)] = gv
        return carry

    lax.fori_loop(0, NBLA, _extract, 0)
    pltpu.sync_copy(gpart.at[pl.ds(0, RTA)], g_sh.at[pl.ds(ra, RTA)])

    def _zden(i, carry):
        den_local[pl.ds(i * LQ, LQ)] = zeros16
        return carry

    lax.fori_loop(0, N // LQ, _zden, 0)
    plsc.subcore_barrier()
    pltpu.sync_copy(g_sh, g_vmem)

    def _fetch(j, buf):
        pltpu.async_copy(ei_hbm.at[0, pl.ds(ebase + j * B, B)],
                         src_ring.at[buf], sem_i)
        idx = dst_all.at[pl.ds(j * B, B)]
        pltpu.async_copy(u_hbm.at[idx], rows_n.at[buf], sem_g)

    def _drain(sem, ref):
        pltpu.make_async_copy(u_hbm.at[pl.ds(0, B)], ref, sem).wait()

    _fetch(0, 0)

    def body(j, carry):
        nxt = j + 1

        @pl.when(nxt < NCHUNK)
        def _prefetch():
            @pl.when(nxt >= NBUF)
            def _free_buf():
                _drain(sem_sc, rows_n.at[0])
            _fetch(nxt, lax.rem(nxt, NBUF))

        _drain(sem_g, rows_n.at[0])
        _drain(sem_i, src_ring.at[0])
        buf = lax.rem(j, NBUF)
        pltpu.async_copy(rows_n.at[buf], accn_sh.at[src_ring.at[buf]],
                         sem_sc, add=True)
        for k in range(B // LQ):
            s16 = src_ring[buf, pl.ds(k * LQ, LQ)]
            d16 = dst_all[pl.ds(j * B + k * LQ, LQ)]
            gv = plsc.load_gather(g_vmem, [d16])
            plsc.addupdate_scatter(den_local, [s16], gv)
        return carry

    lax.fori_loop(0, NCHUNK, body, 0)
    for _ in range(NBUF):
        _drain(sem_sc, rows_n.at[0])
    pltpu.sync_copy(den_local, den_all.at[sid])
    plsc.subcore_barrier()

    for t in range(NS):
        pltpu.sync_copy(den_all.at[t, pl.ds(ra, RTA)],
                        dred.at[t, pl.ds(0, RTA)])

    def _zstage(i, carry):
        den_stage[i, :] = zeros16
        return carry

    lax.fori_loop(0, RTA, _zstage, 0)

    def _reduce(kk, carry):
        acc = zeros16
        for t in range(NS):
            acc = acc + dred[t, pl.ds(kk * LQ, LQ)]
        plsc.store_scatter(den_stage, [iota + kk * LQ, zidx], acc,
                           mask=(iota + kk * LQ) < RTA)
        return carry

    lax.fori_loop(0, NBLA, _reduce, 0)

    pltpu.sync_copy(den_stage, out_den.at[cid, pl.ds(ra, RTA)])
    for k in range(nfull + (1 if rem else 0)):
        r0 = rbase + k * B
        w = min(B, RT - k * B)
        pltpu.sync_copy(accn_sh.at[pl.ds(r0, w)], rows_n.at[0, pl.ds(0, w)])
        pltpu.sync_copy(rows_n.at[0, pl.ds(0, w)],
                        out_num.at[cid, pl.ds(r0, w)])


@functools.lru_cache(maxsize=None)
def _sc_agg():
    return pl.kernel(
        _sc_body,
        out_type=(
            jax.ShapeDtypeStruct((NC, N, OUT_F), jnp.bfloat16),
            jax.ShapeDtypeStruct((NC, N, DG), jnp.float32),
        ),
        mesh=plsc.VectorSubcoreMesh(core_axis_name="c", subcore_axis_name="s"),
        compiler_params=pltpu.CompilerParams(use_tc_tiling_on_sc=False,
                                             needs_layout_passes=False),
        scratch_types=[
            pltpu.VMEM((NBUF, B), jnp.int32),
            pltpu.VMEM((EC,), jnp.int32),
            pltpu.VMEM((NBUF, B, OUT_F), jnp.bfloat16),
            pltpu.VMEM((NBLA * LQ, DG), jnp.float32),
            pltpu.VMEM((NBLA * LQ,), jnp.float32),
            pltpu.VMEM((N,), jnp.float32),
            pltpu.VMEM((N,), jnp.float32),
            pltpu.VMEM((RTA, DG), jnp.float32),
            pltpu.VMEM((NS, NBLA * LQ), jnp.float32),
            pltpu.VMEM_SHARED((N, OUT_F), jnp.bfloat16),
            pltpu.VMEM_SHARED((N,), jnp.float32),
            pltpu.VMEM_SHARED((NS, N), jnp.float32),
            pltpu.SemaphoreType.DMA,
            pltpu.SemaphoreType.DMA,
            pltpu.SemaphoreType.DMA,
        ],
    )


def _tc2_body(num_ref, den_ref, out_ref):
    num = (num_ref[0].astype(jnp.float32) + num_ref[1].astype(jnp.float32))
    den = jnp.sum(den_ref[0] + den_ref[1], axis=1, keepdims=True)
    pos = den > 0.0
    hp = jnp.where(pos, num / jnp.where(pos, den, 1.0), 0.0)
    out_ref[...] = jnp.where(hp > 0.0, hp,
                             jnp.exp(jnp.minimum(hp, 0.0)) - 1.0)


def _tc2(num, den):
    return pl.pallas_call(
        _tc2_body,
        out_shape=jax.ShapeDtypeStruct((N, OUT_F), jnp.float32),
    )(num, den)


def kernel(h, edge_index, W, a):
    a2 = a[OUT_F:, 0][None, :]
    u, g16 = _tc1(h, W, a2)
    zn = jnp.zeros((B, OUT_F), jnp.bfloat16)
    num, den = _sc_agg()(u, g16, edge_index, zn)
    return _tc2(num, den)


# speedup vs baseline: 1.5716x; 1.0212x over previous
import functools

import jax
import jax.numpy as jnp
from jax import lax
from jax.experimental import pallas as pl
from jax.experimental.pallas import tpu as pltpu
from jax.experimental.pallas import tpu_sc as plsc

N = 10000
IN_F = 128
OUT_F = 128
ALPHA = 0.2
E = 320000
DG = 16
NC = 2
NS = 16
NW = NC * NS
EC = E // NW
B = 80
NCHUNK = EC // B
RT = N // NS
RTA = 632
NBLA = (RTA + 15) // 16


def _tc1_body(h_ref, w_ref, a2_ref, u_ref, g16_ref):
    wh = jax.nn.leaky_relu(
        jnp.dot(h_ref[...], w_ref[...], preferred_element_type=jnp.float32),
        negative_slope=ALPHA)
    e2 = jnp.sum(wh * a2_ref[...], axis=1, keepdims=True)
    g = jnp.exp(e2 - jnp.max(e2))
    u_ref[...] = (wh * g).astype(jnp.bfloat16)
    lane = lax.broadcasted_iota(jnp.int32, (N, DG), 1)
    g16_ref[...] = jnp.where(lane == 0, g, 0.0)


def _tc1(h, w, a2):
    return pl.pallas_call(
        _tc1_body,
        out_shape=(
            jax.ShapeDtypeStruct((N, OUT_F), jnp.bfloat16),
            jax.ShapeDtypeStruct((N, DG), jnp.float32),
        ),
    )(h, w, a2)


NBUF = 3
LQ = 16


def _sc_body(u_hbm, g16_hbm, ei_hbm, zn_hbm, out_num, out_den,
             src_ring, dst_all, rows_n, gbuf, gpart, g_vmem, den_local,
             den_stage, dred, accn_sh, g_sh, den_all,
             sem_i, sem_g, sem_sc):
    cid = lax.axis_index("c")
    sid = lax.axis_index("s")
    wid = sid * NC + cid
    ebase = wid * EC
    rbase = sid * RT
    ra = pl.multiple_of(rbase - lax.rem(rbase, 8), 8)
    iota = lax.iota(jnp.int32, LQ)
    zeros16 = jnp.zeros((LQ,), jnp.float32)
    zidx = jnp.zeros((LQ,), jnp.int32)

    pltpu.sync_copy(zn_hbm, rows_n.at[0])
    nfull = RT // B
    for k in range(nfull):
        pltpu.sync_copy(rows_n.at[0], accn_sh.at[pl.ds(rbase + k * B, B)])
    rem = RT - nfull * B
    if rem:
        pltpu.sync_copy(rows_n.at[0, pl.ds(0, rem)],
                        accn_sh.at[pl.ds(rbase + nfull * B, rem)])
    pltpu.sync_copy(ei_hbm.at[1, pl.ds(ebase, EC)], dst_all)

    pltpu.sync_copy(g16_hbm.at[pl.ds(ra, RTA)], gbuf.at[pl.ds(0, RTA)])

    def _extract(kk, carry):
        gv = plsc.load_gather(gbuf, [iota + kk * LQ, zidx])
        gpart[pl.ds(kk * LQ, LQ)] = gv
        return carry

    lax.fori_loop(0, NBLA, _extract, 0)
    pltpu.sync_copy(gpart.at[pl.ds(0, RTA)], g_sh.at[pl.ds(ra, RTA)])

    def _zden(i, carry):
        den_local[pl.ds(i * LQ, LQ)] = zeros16
        return carry

    lax.fori_loop(0, N // LQ, _zden, 0)
    plsc.subcore_barrier()
    pltpu.sync_copy(g_sh, g_vmem)

    def _fetch(j, buf):
        pltpu.async_copy(ei_hbm.at[0, pl.ds(ebase + j * B, B)],
                         src_ring.at[buf], sem_i)
        idx = dst_all.at[pl.ds(j * B, B)]
        pltpu.async_copy(u_hbm.at[idx], rows_n.at[buf], sem_g)

    def _drain(sem, ref):
        pltpu.make_async_copy(u_hbm.at[pl.ds(0, B)], ref, sem).wait()

    _fetch(0, 0)

    def body(j, carry):
        nxt = j + 1

        @pl.when(nxt < NCHUNK)
        def _prefetch():
            @pl.when(nxt >= NBUF)
            def _free_buf():
                _drain(sem_sc, rows_n.at[0])
            _fetch(nxt, lax.rem(nxt, NBUF))

        _drain(sem_g, rows_n.at[0])
        _drain(sem_i, src_ring.at[0])
        buf = lax.rem(j, NBUF)
        pltpu.async_copy(rows_n.at[buf], accn_sh.at[src_ring.at[buf]],
                         sem_sc, add=True)
        for k in range(B // LQ):
            s16 = src_ring[buf, pl.ds(k * LQ, LQ)]
            d16 = dst_all[pl.ds(j * B + k * LQ, LQ)]
            gv = plsc.load_gather(g_vmem, [d16])
            plsc.addupdate_scatter(den_local, [s16], gv)
        return carry

    lax.fori_loop(0, NCHUNK, body, 0)
    for _ in range(NBUF):
        _drain(sem_sc, rows_n.at[0])
    pltpu.sync_copy(den_local, den_all.at[sid])
    plsc.subcore_barrier()

    for t in range(NS):
        pltpu.sync_copy(den_all.at[t, pl.ds(ra, RTA)],
                        dred.at[t, pl.ds(0, RTA)])

    def _zstage(i, carry):
        den_stage[i, :] = zeros16
        return carry

    lax.fori_loop(0, RTA, _zstage, 0)

    def _reduce(kk, carry):
        acc = zeros16
        for t in range(NS):
            acc = acc + dred[t, pl.ds(kk * LQ, LQ)]
        plsc.store_scatter(den_stage, [iota + kk * LQ, zidx], acc,
                           mask=(iota + kk * LQ) < RTA)
        return carry

    lax.fori_loop(0, NBLA, _reduce, 0)

    pltpu.sync_copy(den_stage, out_den.at[cid, pl.ds(ra, RTA)])
    for k in range(nfull + (1 if rem else 0)):
        r0 = rbase + k * B
        w = min(B, RT - k * B)
        pltpu.sync_copy(accn_sh.at[pl.ds(r0, w)], rows_n.at[0, pl.ds(0, w)])
        pltpu.sync_copy(rows_n.at[0, pl.ds(0, w)],
                        out_num.at[cid, pl.ds(r0, w)])


@functools.lru_cache(maxsize=None)
def _sc_agg():
    return pl.kernel(
        _sc_body,
        out_type=(
            jax.ShapeDtypeStruct((NC, N, OUT_F), jnp.bfloat16),
            jax.ShapeDtypeStruct((NC, N, DG), jnp.float32),
        ),
        mesh=plsc.VectorSubcoreMesh(core_axis_name="c", subcore_axis_name="s"),
        compiler_params=pltpu.CompilerParams(use_tc_tiling_on_sc=False,
                                             needs_layout_passes=False),
        scratch_types=[
            pltpu.VMEM((NBUF, B), jnp.int32),
            pltpu.VMEM((EC,), jnp.int32),
            pltpu.VMEM((NBUF, B, OUT_F), jnp.bfloat16),
            pltpu.VMEM((NBLA * LQ, DG), jnp.float32),
            pltpu.VMEM((NBLA * LQ,), jnp.float32),
            pltpu.VMEM((N,), jnp.float32),
            pltpu.VMEM((N,), jnp.float32),
            pltpu.VMEM((RTA, DG), jnp.float32),
            pltpu.VMEM((NS, NBLA * LQ), jnp.float32),
            pltpu.VMEM_SHARED((N, OUT_F), jnp.bfloat16),
            pltpu.VMEM_SHARED((N,), jnp.float32),
            pltpu.VMEM_SHARED((NS, N), jnp.float32),
            pltpu.SemaphoreType.DMA,
            pltpu.SemaphoreType.DMA,
            pltpu.SemaphoreType.DMA,
        ],
    )


def _tc2_body(num_ref, den_ref, out_ref):
    nv = jnp.reshape(num_ref[...], (NC * N, OUT_F))
    num = (nv[:N].astype(jnp.float32) + nv[N:].astype(jnp.float32))
    den = jnp.sum(den_ref[0] + den_ref[1], axis=1, keepdims=True)
    pos = den > 0.0
    hp = jnp.where(pos, num / jnp.where(pos, den, 1.0), 0.0)
    out_ref[...] = jnp.where(hp > 0.0, hp,
                             jnp.exp(jnp.minimum(hp, 0.0)) - 1.0)


def _tc2(num, den):
    return pl.pallas_call(
        _tc2_body,
        out_shape=jax.ShapeDtypeStruct((N, OUT_F), jnp.float32),
    )(num.reshape(-1), den)


def kernel(h, edge_index, W, a):
    a2 = a[OUT_F:, 0][None, :]
    u, g16 = _tc1(h, W, a2)
    zn = jnp.zeros((B, OUT_F), jnp.bfloat16)
    num, den = _sc_agg()(u, g16, edge_index, zn)
    return _tc2(num, den)


# speedup vs baseline: 1.6162x; 1.0283x over previous
import functools

import jax
import jax.numpy as jnp
from jax import lax
from jax.experimental import pallas as pl
from jax.experimental.pallas import tpu as pltpu
from jax.experimental.pallas import tpu_sc as plsc

N = 10000
IN_F = 128
OUT_F = 128
ALPHA = 0.2
E = 320000
DG = 16
NC = 2
NS = 16
NW = NC * NS
EC = E // NW
B = 80
NCHUNK = EC // B
RT = N // NS
RTA = 632
NBLA = (RTA + 15) // 16


def _tc1_body(h_ref, w_ref, a2_ref, u_ref, g16_ref):
    wh = jax.nn.leaky_relu(
        jnp.dot(h_ref[...], w_ref[...], preferred_element_type=jnp.float32),
        negative_slope=ALPHA)
    e2 = jnp.sum(wh * a2_ref[...], axis=1, keepdims=True)
    g = jnp.exp(e2 - jnp.max(e2))
    u_ref[...] = (wh * g).astype(jnp.bfloat16)
    lane = lax.broadcasted_iota(jnp.int32, (N, OUT_F), 1)
    g16_ref[...] = jnp.where(lane == 0, g, 0.0)


def _tc1(h, w, a2):
    return pl.pallas_call(
        _tc1_body,
        out_shape=(
            jax.ShapeDtypeStruct((N, OUT_F), jnp.bfloat16),
            jax.ShapeDtypeStruct((N, OUT_F), jnp.float32),
        ),
    )(h, w, a2)


NBUF = 3
LQ = 16


def _sc_body(u_hbm, g16_hbm, ei_hbm, zn_hbm, out_num, out_den,
             src_ring, dst_all, rows_n, gbuf, gpart, g_vmem, den_local,
             den_stage, dred, accn_sh, g_sh, den_all,
             sem_i, sem_g, sem_sc):
    cid = lax.axis_index("c")
    sid = lax.axis_index("s")
    wid = sid * NC + cid
    ebase = wid * EC
    rbase = sid * RT
    ra = pl.multiple_of(rbase - lax.rem(rbase, 8), 8)
    iota = lax.iota(jnp.int32, LQ)
    zeros16 = jnp.zeros((LQ,), jnp.float32)
    zidx = jnp.zeros((LQ,), jnp.int32)

    pltpu.sync_copy(zn_hbm, rows_n.at[0])
    nfull = RT // B
    for k in range(nfull):
        pltpu.sync_copy(rows_n.at[0], accn_sh.at[pl.ds(rbase + k * B, B)])
    rem = RT - nfull * B
    if rem:
        pltpu.sync_copy(rows_n.at[0, pl.ds(0, rem)],
                        accn_sh.at[pl.ds(rbase + nfull * B, rem)])
    pltpu.sync_copy(ei_hbm.at[1, pl.ds(ebase, EC)], dst_all)

    pltpu.sync_copy(g16_hbm.at[pl.ds(ra, RTA), pl.ds(0, DG)],
                    gbuf.at[pl.ds(0, RTA)])

    def _extract(kk, carry):
        gv = plsc.load_gather(gbuf, [iota + kk * LQ, zidx])
        gpart[pl.ds(kk * LQ, LQ)] = gv
        return carry

    lax.fori_loop(0, NBLA, _extract, 0)
    pltpu.sync_copy(gpart.at[pl.ds(0, RTA)], g_sh.at[pl.ds(ra, RTA)])

    def _zden(i, carry):
        den_local[pl.ds(i * LQ, LQ)] = zeros16
        return carry

    lax.fori_loop(0, N // LQ, _zden, 0)
    plsc.subcore_barrier()
    pltpu.sync_copy(g_sh, g_vmem)

    def _fetch(j, buf):
        pltpu.async_copy(ei_hbm.at[0, pl.ds(ebase + j * B, B)],
                         src_ring.at[buf], sem_i)
        idx = dst_all.at[pl.ds(j * B, B)]
        pltpu.async_copy(u_hbm.at[idx], rows_n.at[buf], sem_g)

    def _drain(sem, ref):
        pltpu.make_async_copy(u_hbm.at[pl.ds(0, B)], ref, sem).wait()

    _fetch(0, 0)

    def body(j, carry):
        nxt = j + 1

        @pl.when(nxt < NCHUNK)
        def _prefetch():
            @pl.when(nxt >= NBUF)
            def _free_buf():
                _drain(sem_sc, rows_n.at[0])
            _fetch(nxt, lax.rem(nxt, NBUF))

        _drain(sem_g, rows_n.at[0])
        _drain(sem_i, src_ring.at[0])
        buf = lax.rem(j, NBUF)
        pltpu.async_copy(rows_n.at[buf], accn_sh.at[src_ring.at[buf]],
                         sem_sc, add=True)
        for k in range(B // LQ):
            s16 = src_ring[buf, pl.ds(k * LQ, LQ)]
            d16 = dst_all[pl.ds(j * B + k * LQ, LQ)]
            gv = plsc.load_gather(g_vmem, [d16])
            plsc.addupdate_scatter(den_local, [s16], gv)
        return carry

    lax.fori_loop(0, NCHUNK, body, 0)
    for _ in range(NBUF):
        _drain(sem_sc, rows_n.at[0])
    pltpu.sync_copy(den_local, den_all.at[sid])
    plsc.subcore_barrier()

    for t in range(NS):
        pltpu.sync_copy(den_all.at[t, pl.ds(ra, RTA)],
                        dred.at[t, pl.ds(0, RTA)])

    def _zstage(i, carry):
        den_stage[i, :] = zeros16
        return carry

    lax.fori_loop(0, RTA, _zstage, 0)

    def _reduce(kk, carry):
        acc = zeros16
        for t in range(NS):
            acc = acc + dred[t, pl.ds(kk * LQ, LQ)]
        plsc.store_scatter(den_stage, [iota + kk * LQ, zidx], acc,
                           mask=(iota + kk * LQ) < RTA)
        return carry

    lax.fori_loop(0, NBLA, _reduce, 0)

    pltpu.sync_copy(den_stage, out_den.at[cid, pl.ds(ra, RTA)])
    for k in range(nfull + (1 if rem else 0)):
        r0 = rbase + k * B
        w = min(B, RT - k * B)
        pltpu.sync_copy(accn_sh.at[pl.ds(r0, w)], rows_n.at[0, pl.ds(0, w)])
        pltpu.sync_copy(rows_n.at[0, pl.ds(0, w)],
                        out_num.at[cid, pl.ds(r0, w)])


@functools.lru_cache(maxsize=None)
def _sc_agg():
    return pl.kernel(
        _sc_body,
        out_type=(
            jax.ShapeDtypeStruct((NC, N, OUT_F), jnp.bfloat16),
            jax.ShapeDtypeStruct((NC, N, DG), jnp.float32),
        ),
        mesh=plsc.VectorSubcoreMesh(core_axis_name="c", subcore_axis_name="s"),
        compiler_params=pltpu.CompilerParams(use_tc_tiling_on_sc=False,
                                             needs_layout_passes=False),
        scratch_types=[
            pltpu.VMEM((NBUF, B), jnp.int32),
            pltpu.VMEM((EC,), jnp.int32),
            pltpu.VMEM((NBUF, B, OUT_F), jnp.bfloat16),
            pltpu.VMEM((NBLA * LQ, DG), jnp.float32),
            pltpu.VMEM((NBLA * LQ,), jnp.float32),
            pltpu.VMEM((N,), jnp.float32),
            pltpu.VMEM((N,), jnp.float32),
            pltpu.VMEM((RTA, DG), jnp.float32),
            pltpu.VMEM((NS, NBLA * LQ), jnp.float32),
            pltpu.VMEM_SHARED((N, OUT_F), jnp.bfloat16),
            pltpu.VMEM_SHARED((N,), jnp.float32),
            pltpu.VMEM_SHARED((NS, N), jnp.float32),
            pltpu.SemaphoreType.DMA,
            pltpu.SemaphoreType.DMA,
            pltpu.SemaphoreType.DMA,
        ],
    )


def _tc2_body(num_ref, den_ref, out_ref):
    nv = jnp.reshape(num_ref[...], (NC * N, OUT_F))
    num = (nv[:N].astype(jnp.float32) + nv[N:].astype(jnp.float32))
    den = jnp.sum(den_ref[0] + den_ref[1], axis=1, keepdims=True)
    pos = den > 0.0
    hp = jnp.where(pos, num / jnp.where(pos, den, 1.0), 0.0)
    out_ref[...] = jnp.where(hp > 0.0, hp,
                             jnp.exp(jnp.minimum(hp, 0.0)) - 1.0)


def _tc2(num, den):
    return pl.pallas_call(
        _tc2_body,
        out_shape=jax.ShapeDtypeStruct((N, OUT_F), jnp.float32),
    )(num.reshape(-1), den)


def kernel(h, edge_index, W, a):
    a2 = a[OUT_F:, 0][None, :]
    u, g16 = _tc1(h, W, a2)
    zn = jnp.zeros((B, OUT_F), jnp.bfloat16)
    num, den = _sc_agg()(u, g16, edge_index, zn)
    return _tc2(num, den)
